# baseline (speedup 1.0000x reference)
import jax
import jax.numpy as jnp
from jax import lax
from jax.experimental import pallas as pl
from jax.experimental.pallas import tpu as pltpu
from jax.experimental.pallas import tpu_sc as plsc

U_NUM = 25000
I_NUM = 25000
N = U_NUM + I_NUM
E = 800000
D = 64
HALF = 32
N_LAYERS = 3

NC = 2
NS = 16
CHUNK = 384
SUB = 128
NSUB = CHUNK // SUB
CHUNKS_PER_TILE = 132
E_PAD = CHUNKS_PER_TILE * CHUNK * NS
N_PAD = 51200
ROWS_PER_TILE = N_PAD // NS
MEAN_PART = 320
MEAN_NPART = ROWS_PER_TILE // MEAN_PART


def _sfa_body(ego0, srcm, dstm, w, zeros, mean_out, l1, l2, l3,
              src_v0, dst_v0, w_v0, rows0, src_v1, dst_v1, w_v1, rows1,
              acc, gsem0, ssem0, gsem1, ssem1):
    bufs = ((src_v0, dst_v0, w_v0, rows0, gsem0, ssem0),
            (src_v1, dst_v1, w_v1, rows1, gsem1, ssem1))
    c = lax.axis_index("c")
    t = lax.axis_index("s")
    r0 = t * ROWS_PER_TILE
    K = CHUNKS_PER_TILE

    layer_bufs = [ego0, l1, l2, l3]
    for layer in range(N_LAYERS):
        cur = layer_bufs[layer]
        nxt = layer_bufs[layer + 1]

        def load_and_fire(j, b, cur=cur):
            src_v, dst_v, w_v, rows_v, gsem, _ = bufs[b]
            base = t * K + j
            pltpu.sync_copy(srcm.at[pl.ds(base * NSUB, NSUB)], src_v)
            pltpu.sync_copy(dstm.at[pl.ds(base * NSUB, NSUB)], dst_v)
            pltpu.sync_copy(w.at[pl.ds(base * CHUNK, CHUNK)], w_v)
            for s in range(NSUB):
                pltpu.async_copy(cur.at[c].at[src_v.at[s]],
                                 rows_v.at[pl.ds(s * SUB, SUB)], gsem)

        def wait_gathers(b, cur=cur):
            src_v, _, _, rows_v, gsem, _ = bufs[b]
            for s in range(NSUB):
                pltpu.make_async_copy(cur.at[c].at[src_v.at[s]],
                                      rows_v.at[pl.ds(s * SUB, SUB)],
                                      gsem).wait()

        def multiply(b):
            _, _, w_v, rows_v, _, _ = bufs[b]

            def mul_body(g, _):
                e = g * 16
                wv = w_v[pl.ds(e, 16)]
                for i in range(16):
                    ws = jnp.take_along_axis(
                        wv, jnp.full((16,), i, jnp.int32), axis=0)
                    rows_v[e + i, pl.ds(0, 16)] = rows_v[e + i, pl.ds(0, 16)] * ws
                    rows_v[e + i, pl.ds(16, 16)] = rows_v[e + i, pl.ds(16, 16)] * ws
                return 0
            lax.fori_loop(0, CHUNK // 16, mul_body, 0)

        def fire_scatter(b):
            _, dst_v, _, rows_v, _, ssem = bufs[b]
            for s in range(NSUB):
                pltpu.async_copy(rows_v.at[pl.ds(s * SUB, SUB)],
                                 acc.at[dst_v.at[s]], ssem, add=True)

        def wait_scatter(b):
            _, dst_v, _, rows_v, _, ssem = bufs[b]
            for s in range(NSUB):
                pltpu.make_async_copy(rows_v.at[pl.ds(s * SUB, SUB)],
                                      acc.at[dst_v.at[s]], ssem).wait()

        with jax.named_scope(f"zero{layer}"):
            pltpu.sync_copy(zeros.at[pl.ds(r0, ROWS_PER_TILE)],
                            acc.at[pl.ds(r0, ROWS_PER_TILE)])
            plsc.subcore_barrier()

        load_and_fire(0, 0)
        load_and_fire(1, 1)
        wait_gathers(0)
        multiply(0)
        fire_scatter(0)

        with jax.named_scope(f"edges{layer}"):
            @pl.loop(1, K - 1, step=2)
            def _(k):
                for b01 in range(2):
                    j = k + b01
                    b = (1 + b01) % 2
                    wait_scatter(1 - b)
                    load_and_fire(j + 1, 1 - b)
                    wait_gathers(b)
                    multiply(b)
                    fire_scatter(b)

        wait_scatter(0)
        wait_gathers(1)
        multiply(1)
        fire_scatter(1)
        wait_scatter(1)
        plsc.subcore_barrier()

        with jax.named_scope(f"publish{layer}"):
            pltpu.sync_copy(acc.at[pl.ds(r0, ROWS_PER_TILE)],
                            nxt.at[c].at[pl.ds(r0, ROWS_PER_TILE)])

    plsc.subcore_barrier()

    with jax.named_scope("meanpass"):
        bufa = rows0.at[pl.ds(0, MEAN_PART)]
        bufb = rows1.at[pl.ds(0, MEAN_PART)]
        for p in range(MEAN_NPART):
            q0 = r0 + p * MEAN_PART
            pltpu.sync_copy(ego0.at[c].at[pl.ds(q0, MEAN_PART)], bufa)
            for li, lbuf in enumerate((l1, l2, l3)):
                pltpu.sync_copy(lbuf.at[c].at[pl.ds(q0, MEAN_PART)], bufb)
                last = li == 2

                def add_body(i, _, last=last):
                    for off in (0, 16):
                        v = bufa[i, pl.ds(off, 16)] + bufb[i, pl.ds(off, 16)]
                        if last:
                            v = v * 0.25
                        bufa[i, pl.ds(off, 16)] = v
                    return 0
                lax.fori_loop(0, MEAN_PART, add_body, 0)
            pltpu.sync_copy(bufa, mean_out.at[c].at[pl.ds(q0, MEAN_PART)])


@jax.jit
def _sfa_encoder(user_emb, item_emb, edge_index, edge_values):
    ego0 = jnp.concatenate(
        [user_emb, item_emb, jnp.zeros((N_PAD - N, D), jnp.float32)], axis=0)
    ego0_h = ego0.reshape(N_PAD, NC, HALF).transpose(1, 0, 2)
    pad = E_PAD - E
    src_p = jnp.concatenate([edge_index[0], jnp.zeros((pad,), jnp.int32)])
    dst_p = jnp.concatenate([edge_index[1], jnp.zeros((pad,), jnp.int32)])
    w_p = jnp.concatenate([edge_values, jnp.zeros((pad,), jnp.float32)])
    srcm = src_p.reshape(-1, SUB)
    dstm = dst_p.reshape(-1, SUB)
    zeros = jnp.zeros((N_PAD, HALF), jnp.float32)

    mesh = plsc.VectorSubcoreMesh(core_axis_name="c", subcore_axis_name="s")
    f32 = jnp.float32
    out_type = tuple(jax.ShapeDtypeStruct((NC, N_PAD, HALF), f32) for _ in range(4))
    dbuf = [
        pltpu.VMEM((NSUB, SUB), jnp.int32),
        pltpu.VMEM((NSUB, SUB), jnp.int32),
        pltpu.VMEM((CHUNK,), f32),
        pltpu.VMEM((CHUNK, HALF), f32),
    ]
    kern = pl.kernel(
        _sfa_body,
        out_type=out_type,
        mesh=mesh,
        scratch_types=dbuf + dbuf + [
            pltpu.VMEM_SHARED((N_PAD, HALF), f32),
            pltpu.SemaphoreType.DMA,
            pltpu.SemaphoreType.DMA,
            pltpu.SemaphoreType.DMA,
            pltpu.SemaphoreType.DMA,
        ],
        compiler_params=pltpu.CompilerParams(use_tc_tiling_on_sc=False),
    )
    mean_h, _, _, _ = kern(ego0_h, srcm, dstm, w_p, zeros)
    mean = mean_h.transpose(1, 0, 2).reshape(N_PAD, D)
    return mean[:U_NUM], mean[U_NUM:N]


def kernel(user_emb, item_emb, edge_index, edge_values):
    return _sfa_encoder(user_emb, item_emb, edge_index, edge_values)


# speedup vs baseline: 1.2750x; 1.2750x over previous
import jax
import jax.numpy as jnp
from jax import lax
from jax.experimental import pallas as pl
from jax.experimental.pallas import tpu as pltpu
from jax.experimental.pallas import tpu_sc as plsc

U_NUM = 25000
I_NUM = 25000
N = U_NUM + I_NUM
E = 800000
D = 64
HALF = 32
N_LAYERS = 3

NC = 2
NS = 16
CHUNK = 384
SUB = 128
NSUB = CHUNK // SUB
PACK_ROWS = 3 * NSUB
CHUNKS_PER_TILE = 132
E_PAD = CHUNKS_PER_TILE * CHUNK * NS
N_PAD = 50048
ROWS_PER_TILE = N_PAD // NS
MEAN_PART = 136
MEAN_NPART = ROWS_PER_TILE // MEAN_PART
NIDX = 4


def _sfa_body(ego0, pack, zeros, mean_out, l1, l2, l3,
              i0, i1, i2, i3, rows0, rows1,
              acc, is0, is1, is2, is3, gsem0, gsem1, ssem0, ssem1):
    idxb = (i0, i1, i2, i3)
    isem = (is0, is1, is2, is3)
    rows = (rows0, rows1)
    gsem = (gsem0, gsem1)
    ssem = (ssem0, ssem1)
    c = lax.axis_index("c")
    t = lax.axis_index("s")
    r0 = t * ROWS_PER_TILE
    K = CHUNKS_PER_TILE

    layer_bufs = [ego0, l1, l2, l3]
    for layer in range(N_LAYERS):
        cur = layer_bufs[layer]
        nxt = layer_bufs[layer + 1]

        def fire_idx(j, r):
            pltpu.async_copy(pack.at[t * K + j], idxb[r], isem[r])

        def wait_idx(j, r):
            pltpu.make_async_copy(pack.at[t * K + j], idxb[r], isem[r]).wait()

        def fire_gathers(j, b, r, cur=cur):
            for s in range(NSUB):
                pltpu.async_copy(cur.at[c].at[idxb[r].at[s]],
                                 rows[b].at[pl.ds(s * SUB, SUB)], gsem[b])

        def wait_gathers(b, r, cur=cur):
            for s in range(NSUB):
                pltpu.make_async_copy(cur.at[c].at[idxb[r].at[s]],
                                      rows[b].at[pl.ds(s * SUB, SUB)],
                                      gsem[b]).wait()

        def multiply(b, r):
            rows_v = rows[b]

            def mul_body(g, _):
                wv = plsc.bitcast(
                    idxb[r][2 * NSUB + g // 8, pl.ds((g % 8) * 16, 16)],
                    jnp.float32)
                e = g * 16
                for i in range(16):
                    ws = jnp.take_along_axis(
                        wv, jnp.full((16,), i, jnp.int32), axis=0)
                    rows_v[e + i, pl.ds(0, 16)] = rows_v[e + i, pl.ds(0, 16)] * ws
                    rows_v[e + i, pl.ds(16, 16)] = rows_v[e + i, pl.ds(16, 16)] * ws
                return 0
            lax.fori_loop(0, CHUNK // 16, mul_body, 0)

        def fire_scatter(b, r):
            for s in range(NSUB):
                pltpu.async_copy(rows[b].at[pl.ds(s * SUB, SUB)],
                                 acc.at[idxb[r].at[NSUB + s]], ssem[b],
                                 add=True)

        def wait_scatter(b, r):
            for s in range(NSUB):
                pltpu.make_async_copy(rows[b].at[pl.ds(s * SUB, SUB)],
                                      acc.at[idxb[r].at[NSUB + s]],
                                      ssem[b]).wait()

        pltpu.sync_copy(zeros.at[pl.ds(r0, ROWS_PER_TILE)],
                        acc.at[pl.ds(r0, ROWS_PER_TILE)])
        plsc.subcore_barrier()

        fire_idx(0, 0)
        fire_idx(1, 1)
        fire_idx(2, 2)
        wait_idx(0, 0)
        fire_gathers(0, 0, 0)
        fire_idx(3, 3)
        wait_idx(1, 1)
        fire_gathers(1, 1, 1)
        wait_gathers(0, 0)
        multiply(0, 0)
        fire_scatter(0, 0)

        @pl.loop(1, K - 3, step=4)
        def _(k):
            for b01 in range(4):
                j = k + b01
                sl = (1 + b01) % NIDX
                b = (1 + b01) % 2
                wait_scatter(1 - b, (sl - 1) % NIDX)
                fire_idx(j + 3, (sl + 3) % NIDX)
                wait_idx(j + 1, (sl + 1) % NIDX)
                fire_gathers(j + 1, 1 - b, (sl + 1) % NIDX)
                wait_gathers(b, sl)
                multiply(b, sl)
                fire_scatter(b, sl)

        for j in (K - 3, K - 2, K - 1):
            sl = j % NIDX
            b = j % 2
            wait_scatter(1 - b, (sl - 1) % NIDX)
            if j + 1 < K:
                wait_idx(j + 1, (sl + 1) % NIDX)
                fire_gathers(j + 1, 1 - b, (sl + 1) % NIDX)
            wait_gathers(b, sl)
            multiply(b, sl)
            fire_scatter(b, sl)
        wait_scatter((K - 1) % 2, (K - 1) % NIDX)
        plsc.subcore_barrier()

        pltpu.sync_copy(acc.at[pl.ds(r0, ROWS_PER_TILE)],
                        nxt.at[c].at[pl.ds(r0, ROWS_PER_TILE)])

    plsc.subcore_barrier()

    bufa = rows0.at[pl.ds(0, MEAN_PART)]
    bufb = rows1.at[pl.ds(0, MEAN_PART)]
    for p in range(MEAN_NPART):
        q0 = r0 + p * MEAN_PART
        pltpu.sync_copy(ego0.at[c].at[pl.ds(q0, MEAN_PART)], bufa)
        for li, lbuf in enumerate((l1, l2, l3)):
            pltpu.sync_copy(lbuf.at[c].at[pl.ds(q0, MEAN_PART)], bufb)
            last = li == 2

            def add_body(i, _, last=last):
                for off in (0, 16):
                    v = bufa[i, pl.ds(off, 16)] + bufb[i, pl.ds(off, 16)]
                    if last:
                        v = v * 0.25
                    bufa[i, pl.ds(off, 16)] = v
                return 0
            lax.fori_loop(0, MEAN_PART, add_body, 0)
        pltpu.sync_copy(bufa, mean_out.at[c].at[pl.ds(q0, MEAN_PART)])


@jax.jit
def _sfa_encoder(user_emb, item_emb, edge_index, edge_values):
    ego0 = jnp.concatenate(
        [user_emb, item_emb, jnp.zeros((N_PAD - N, D), jnp.float32)], axis=0)
    ego0_h = ego0.reshape(N_PAD, NC, HALF).transpose(1, 0, 2)
    pad = E_PAD - E
    src_p = jnp.concatenate([edge_index[0], jnp.zeros((pad,), jnp.int32)])
    dst_p = jnp.concatenate([edge_index[1], jnp.zeros((pad,), jnp.int32)])
    w_p = jnp.concatenate([edge_values, jnp.zeros((pad,), jnp.float32)])
    w_bits = lax.bitcast_convert_type(w_p, jnp.int32)
    pack = jnp.concatenate(
        [src_p.reshape(-1, NSUB, SUB), dst_p.reshape(-1, NSUB, SUB),
         w_bits.reshape(-1, NSUB, SUB)], axis=1)
    zeros = jnp.zeros((N_PAD, HALF), jnp.float32)

    mesh = plsc.VectorSubcoreMesh(core_axis_name="c", subcore_axis_name="s")
    f32 = jnp.float32
    out_type = tuple(jax.ShapeDtypeStruct((NC, N_PAD, HALF), f32) for _ in range(4))
    kern = pl.kernel(
        _sfa_body,
        out_type=out_type,
        mesh=mesh,
        scratch_types=[pltpu.VMEM((PACK_ROWS, SUB), jnp.int32)] * NIDX + [
            pltpu.VMEM((CHUNK, HALF), f32),
            pltpu.VMEM((CHUNK, HALF), f32),
            pltpu.VMEM_SHARED((N_PAD, HALF), f32),
        ] + [pltpu.SemaphoreType.DMA] * (NIDX + 4),
        compiler_params=pltpu.CompilerParams(use_tc_tiling_on_sc=False,
                                             needs_layout_passes=False),
    )
    mean_h, _, _, _ = kern(ego0_h, pack, zeros)
    mean = mean_h.transpose(1, 0, 2).reshape(N_PAD, D)
    return mean[:U_NUM], mean[U_NUM:N]


def kernel(user_emb, item_emb, edge_index, edge_values):
    return _sfa_encoder(user_emb, item_emb, edge_index, edge_values)


# speedup vs baseline: 1.2863x; 1.0089x over previous
import jax
import jax.numpy as jnp
from jax import lax
from jax.experimental import pallas as pl
from jax.experimental.pallas import tpu as pltpu
from jax.experimental.pallas import tpu_sc as plsc

U_NUM = 25000
I_NUM = 25000
N = U_NUM + I_NUM
E = 800000
D = 64
HALF = 32
N_LAYERS = 3

NC = 2
NS = 16
CHUNK = 384
SUB = 128
NSUB = CHUNK // SUB
PACK_ROWS = 3 * NSUB
CHUNKS_PER_TILE = 132
E_PAD = CHUNKS_PER_TILE * CHUNK * NS
N_PAD = 50048
ROWS_PER_TILE = N_PAD // NS
MEAN_PART = 136
MEAN_NPART = ROWS_PER_TILE // MEAN_PART
NIDX = 4


def _sfa_body(ego0, pack, zeros, mean_out, l1, l2, l3,
              i0, i1, i2, i3, rows0, rows1,
              acc, is0, is1, is2, is3, gsem0, gsem1, ssem0, ssem1):
    idxb = (i0, i1, i2, i3)
    isem = (is0, is1, is2, is3)
    rows = (rows0, rows1)
    gsem = (gsem0, gsem1)
    ssem = (ssem0, ssem1)
    c = lax.axis_index("c")
    t = lax.axis_index("s")
    r0 = t * ROWS_PER_TILE
    K = CHUNKS_PER_TILE

    layer_bufs = [ego0, l1, l2, l3]
    for layer in range(N_LAYERS):
        cur = layer_bufs[layer]
        nxt = layer_bufs[layer + 1]

        def fire_idx(j, r):
            pltpu.async_copy(pack.at[t * K + j], idxb[r], isem[r])

        def wait_idx(j, r):
            pltpu.make_async_copy(pack.at[t * K + j], idxb[r], isem[r]).wait()

        def fire_gathers(j, b, r, cur=cur):
            pltpu.async_copy(cur.at[c].at[idxb[r].at[pl.ds(0, CHUNK)]],
                             rows[b], gsem[b])

        def wait_gathers(b, r, cur=cur):
            pltpu.make_async_copy(cur.at[c].at[idxb[r].at[pl.ds(0, CHUNK)]],
                                  rows[b], gsem[b]).wait()

        def multiply(b, r):
            rows_v = rows[b]

            def mul_body(g, _):
                wv = plsc.bitcast(
                    idxb[r][pl.ds(2 * CHUNK + g * 16, 16)], jnp.float32)
                e = g * 16
                for i in range(16):
                    ws = jnp.take_along_axis(
                        wv, jnp.full((16,), i, jnp.int32), axis=0)
                    rows_v[e + i, pl.ds(0, 16)] = rows_v[e + i, pl.ds(0, 16)] * ws
                    rows_v[e + i, pl.ds(16, 16)] = rows_v[e + i, pl.ds(16, 16)] * ws
                return 0
            lax.fori_loop(0, CHUNK // 16, mul_body, 0)

        def fire_scatter(b, r):
            pltpu.async_copy(rows[b],
                             acc.at[idxb[r].at[pl.ds(CHUNK, CHUNK)]], ssem[b],
                             add=True)

        def wait_scatter(b, r):
            pltpu.make_async_copy(rows[b],
                                  acc.at[idxb[r].at[pl.ds(CHUNK, CHUNK)]],
                                  ssem[b]).wait()

        pltpu.sync_copy(zeros.at[pl.ds(r0, ROWS_PER_TILE)],
                        acc.at[pl.ds(r0, ROWS_PER_TILE)])
        plsc.subcore_barrier()

        fire_idx(0, 0)
        fire_idx(1, 1)
        fire_idx(2, 2)
        wait_idx(0, 0)
        fire_gathers(0, 0, 0)
        fire_idx(3, 3)
        wait_idx(1, 1)
        fire_gathers(1, 1, 1)
        wait_gathers(0, 0)
        multiply(0, 0)
        fire_scatter(0, 0)

        @pl.loop(1, K - 3, step=4)
        def _(k):
            for b01 in range(4):
                j = k + b01
                sl = (1 + b01) % NIDX
                b = (1 + b01) % 2
                wait_scatter(1 - b, (sl - 1) % NIDX)
                fire_idx(j + 3, (sl + 3) % NIDX)
                wait_idx(j + 1, (sl + 1) % NIDX)
                fire_gathers(j + 1, 1 - b, (sl + 1) % NIDX)
                wait_gathers(b, sl)
                multiply(b, sl)
                fire_scatter(b, sl)

        for j in (K - 3, K - 2, K - 1):
            sl = j % NIDX
            b = j % 2
            wait_scatter(1 - b, (sl - 1) % NIDX)
            if j + 1 < K:
                wait_idx(j + 1, (sl + 1) % NIDX)
                fire_gathers(j + 1, 1 - b, (sl + 1) % NIDX)
            wait_gathers(b, sl)
            multiply(b, sl)
            fire_scatter(b, sl)
        wait_scatter((K - 1) % 2, (K - 1) % NIDX)
        plsc.subcore_barrier()

        pltpu.sync_copy(acc.at[pl.ds(r0, ROWS_PER_TILE)],
                        nxt.at[c].at[pl.ds(r0, ROWS_PER_TILE)])

    plsc.subcore_barrier()

    bufa = rows0.at[pl.ds(0, MEAN_PART)]
    bufb = rows1.at[pl.ds(0, MEAN_PART)]
    for p in range(MEAN_NPART):
        q0 = r0 + p * MEAN_PART
        pltpu.sync_copy(ego0.at[c].at[pl.ds(q0, MEAN_PART)], bufa)
        for li, lbuf in enumerate((l1, l2, l3)):
            pltpu.sync_copy(lbuf.at[c].at[pl.ds(q0, MEAN_PART)], bufb)
            last = li == 2

            def add_body(i, _, last=last):
                for off in (0, 16):
                    v = bufa[i, pl.ds(off, 16)] + bufb[i, pl.ds(off, 16)]
                    if last:
                        v = v * 0.25
                    bufa[i, pl.ds(off, 16)] = v
                return 0
            lax.fori_loop(0, MEAN_PART, add_body, 0)
        pltpu.sync_copy(bufa, mean_out.at[c].at[pl.ds(q0, MEAN_PART)])


@jax.jit
def _sfa_encoder(user_emb, item_emb, edge_index, edge_values):
    ego0 = jnp.concatenate(
        [user_emb, item_emb, jnp.zeros((N_PAD - N, D), jnp.float32)], axis=0)
    ego0_h = ego0.reshape(N_PAD, NC, HALF).transpose(1, 0, 2)
    pad = E_PAD - E
    src_p = jnp.concatenate([edge_index[0], jnp.zeros((pad,), jnp.int32)])
    dst_p = jnp.concatenate([edge_index[1], jnp.zeros((pad,), jnp.int32)])
    w_p = jnp.concatenate([edge_values, jnp.zeros((pad,), jnp.float32)])
    w_bits = lax.bitcast_convert_type(w_p, jnp.int32)
    pack = jnp.concatenate(
        [src_p.reshape(-1, CHUNK), dst_p.reshape(-1, CHUNK),
         w_bits.reshape(-1, CHUNK)], axis=1)
    zeros = jnp.zeros((N_PAD, HALF), jnp.float32)

    mesh = plsc.VectorSubcoreMesh(core_axis_name="c", subcore_axis_name="s")
    f32 = jnp.float32
    out_type = tuple(jax.ShapeDtypeStruct((NC, N_PAD, HALF), f32) for _ in range(4))
    kern = pl.kernel(
        _sfa_body,
        out_type=out_type,
        mesh=mesh,
        scratch_types=[pltpu.VMEM((3 * CHUNK,), jnp.int32)] * NIDX + [
            pltpu.VMEM((CHUNK, HALF), f32),
            pltpu.VMEM((CHUNK, HALF), f32),
            pltpu.VMEM_SHARED((N_PAD, HALF), f32),
        ] + [pltpu.SemaphoreType.DMA] * (NIDX + 4),
        compiler_params=pltpu.CompilerParams(use_tc_tiling_on_sc=False,
                                             needs_layout_passes=False),
    )
    mean_h, _, _, _ = kern(ego0_h, pack, zeros)
    mean = mean_h.transpose(1, 0, 2).reshape(N_PAD, D)
    return mean[:U_NUM], mean[U_NUM:N]


def kernel(user_emb, item_emb, edge_index, edge_values):
    return _sfa_encoder(user_emb, item_emb, edge_index, edge_values)


# speedup vs baseline: 1.3720x; 1.0666x over previous
import jax
import jax.numpy as jnp
from jax import lax
from jax.experimental import pallas as pl
from jax.experimental.pallas import tpu as pltpu
from jax.experimental.pallas import tpu_sc as plsc

U_NUM = 25000
I_NUM = 25000
N = U_NUM + I_NUM
E = 800000
D = 64
HALF = 32
N_LAYERS = 3

NC = 2
NS = 16
CHUNK = 384
SUB = 128
NSUB = CHUNK // SUB
PACK_ROWS = 3 * NSUB
CHUNKS_PER_TILE = 132
E_PAD = CHUNKS_PER_TILE * CHUNK * NS
N_PAD = 50048
ROWS_PER_TILE = N_PAD // NS
MEAN_PART = 136
MEAN_NPART = ROWS_PER_TILE // MEAN_PART
NIDX = 4


def _sfa_body(ego0, pack, zeros, l1, l2, l3,
              i0, i1, i2, i3, rows0, rows1,
              acc, is0, is1, is2, is3, gsem0, gsem1, ssem0, ssem1):
    idxb = (i0, i1, i2, i3)
    isem = (is0, is1, is2, is3)
    rows = (rows0, rows1)
    gsem = (gsem0, gsem1)
    ssem = (ssem0, ssem1)
    c = lax.axis_index("c")
    t = lax.axis_index("s")
    r0 = t * ROWS_PER_TILE
    K = CHUNKS_PER_TILE

    layer_bufs = [ego0, l1, l2, l3]
    for layer in range(N_LAYERS):
        cur = layer_bufs[layer]
        nxt = layer_bufs[layer + 1]

        def fire_idx(j, r):
            pltpu.async_copy(pack.at[t * K + j], idxb[r], isem[r])

        def wait_idx(j, r):
            pltpu.make_async_copy(pack.at[t * K + j], idxb[r], isem[r]).wait()

        def fire_gathers(j, b, r, cur=cur):
            pltpu.async_copy(cur.at[c].at[idxb[r].at[pl.ds(0, CHUNK)]],
                             rows[b], gsem[b])

        def wait_gathers(b, r, cur=cur):
            pltpu.make_async_copy(cur.at[c].at[idxb[r].at[pl.ds(0, CHUNK)]],
                                  rows[b], gsem[b]).wait()

        def multiply(b, r):
            rows_v = rows[b]

            def mul_body(g, _):
                wv = plsc.bitcast(
                    idxb[r][pl.ds(2 * CHUNK + g * 16, 16)], jnp.float32)
                e = g * 16
                for i in range(16):
                    ws = jnp.take_along_axis(
                        wv, jnp.full((16,), i, jnp.int32), axis=0)
                    rows_v[e + i, pl.ds(0, 16)] = rows_v[e + i, pl.ds(0, 16)] * ws
                    rows_v[e + i, pl.ds(16, 16)] = rows_v[e + i, pl.ds(16, 16)] * ws
                return 0
            lax.fori_loop(0, CHUNK // 16, mul_body, 0)

        def fire_scatter(b, r):
            pltpu.async_copy(rows[b],
                             acc.at[idxb[r].at[pl.ds(CHUNK, CHUNK)]], ssem[b],
                             add=True)

        def wait_scatter(b, r):
            pltpu.make_async_copy(rows[b],
                                  acc.at[idxb[r].at[pl.ds(CHUNK, CHUNK)]],
                                  ssem[b]).wait()

        pltpu.sync_copy(zeros.at[pl.ds(r0, ROWS_PER_TILE)],
                        acc.at[pl.ds(r0, ROWS_PER_TILE)])
        plsc.subcore_barrier()

        fire_idx(0, 0)
        fire_idx(1, 1)
        fire_idx(2, 2)
        wait_idx(0, 0)
        fire_gathers(0, 0, 0)
        fire_idx(3, 3)
        wait_idx(1, 1)
        fire_gathers(1, 1, 1)
        wait_gathers(0, 0)
        multiply(0, 0)
        fire_scatter(0, 0)

        @pl.loop(1, K - 3, step=4)
        def _(k):
            for b01 in range(4):
                j = k + b01
                sl = (1 + b01) % NIDX
                b = (1 + b01) % 2
                wait_scatter(1 - b, (sl - 1) % NIDX)
                fire_idx(j + 3, (sl + 3) % NIDX)
                wait_idx(j + 1, (sl + 1) % NIDX)
                fire_gathers(j + 1, 1 - b, (sl + 1) % NIDX)
                wait_gathers(b, sl)
                multiply(b, sl)
                fire_scatter(b, sl)

        for j in (K - 3, K - 2, K - 1):
            sl = j % NIDX
            b = j % 2
            wait_scatter(1 - b, (sl - 1) % NIDX)
            if j + 1 < K:
                wait_idx(j + 1, (sl + 1) % NIDX)
                fire_gathers(j + 1, 1 - b, (sl + 1) % NIDX)
            wait_gathers(b, sl)
            multiply(b, sl)
            fire_scatter(b, sl)
        wait_scatter((K - 1) % 2, (K - 1) % NIDX)
        plsc.subcore_barrier()

        pltpu.sync_copy(acc.at[pl.ds(r0, ROWS_PER_TILE)],
                        nxt.at[c].at[pl.ds(r0, ROWS_PER_TILE)])


_X = N_PAD * HALF // 128
_BRX = 544


def _tc_mean_body(a, b, c, d, o):
    o[...] = (a[...] + b[...] + c[...] + d[...]) * 0.25


def _tc_mean(a, b, c, d):
    spec = pl.BlockSpec((1, _BRX, 128), lambda h, r: (h, r, 0))
    return pl.pallas_call(
        _tc_mean_body,
        grid=(NC, _X // _BRX),
        in_specs=[spec] * 4,
        out_specs=spec,
        out_shape=jax.ShapeDtypeStruct((NC, _X, 128), jnp.float32),
    )(a, b, c, d)


@jax.jit
def _sfa_encoder(user_emb, item_emb, edge_index, edge_values):
    ego0 = jnp.concatenate(
        [user_emb, item_emb, jnp.zeros((N_PAD - N, D), jnp.float32)], axis=0)
    ego0_h = ego0.reshape(N_PAD, NC, HALF).transpose(1, 0, 2)
    pad = E_PAD - E
    src_p = jnp.concatenate([edge_index[0], jnp.zeros((pad,), jnp.int32)])
    dst_p = jnp.concatenate([edge_index[1], jnp.zeros((pad,), jnp.int32)])
    w_p = jnp.concatenate([edge_values, jnp.zeros((pad,), jnp.float32)])
    w_bits = lax.bitcast_convert_type(w_p, jnp.int32)
    pack = jnp.concatenate(
        [src_p.reshape(-1, CHUNK), dst_p.reshape(-1, CHUNK),
         w_bits.reshape(-1, CHUNK)], axis=1)
    zeros = jnp.zeros((N_PAD, HALF), jnp.float32)

    mesh = plsc.VectorSubcoreMesh(core_axis_name="c", subcore_axis_name="s")
    f32 = jnp.float32
    out_type = tuple(jax.ShapeDtypeStruct((NC, N_PAD, HALF), f32) for _ in range(3))
    kern = pl.kernel(
        _sfa_body,
        out_type=out_type,
        mesh=mesh,
        scratch_types=[pltpu.VMEM((3 * CHUNK,), jnp.int32)] * NIDX + [
            pltpu.VMEM((CHUNK, HALF), f32),
            pltpu.VMEM((CHUNK, HALF), f32),
            pltpu.VMEM_SHARED((N_PAD, HALF), f32),
        ] + [pltpu.SemaphoreType.DMA] * (NIDX + 4),
        compiler_params=pltpu.CompilerParams(use_tc_tiling_on_sc=False,
                                             needs_layout_passes=False),
    )
    l1, l2, l3 = kern(ego0_h, pack, zeros)
    flat = lambda x: x.reshape(NC, -1, 128)
    mean_h = _tc_mean(flat(ego0_h), flat(l1), flat(l2), flat(l3))
    mean = mean_h.reshape(NC, N_PAD, HALF).transpose(1, 0, 2).reshape(N_PAD, D)
    return mean[:U_NUM], mean[U_NUM:N]


def kernel(user_emb, item_emb, edge_index, edge_values):
    return _sfa_encoder(user_emb, item_emb, edge_index, edge_values)
